# baseline (speedup 1.0000x reference)
import functools

import jax
import jax.numpy as jnp
from jax import lax
from jax.experimental import pallas as pl
from jax.experimental.pallas import tpu as pltpu
from jax.experimental.pallas import tpu_sc as plsc

_D = 64
_L = 20
_LP = 24
_ALPHA = 0.2
_NB = 32
_CH = 512
_SENT = 0x7FFFFFFF


def _gather_rows(table, idx, n_sessions):
    info = plsc.get_sparse_core_info()
    nc, ns = info.num_cores, info.num_subcores
    nw = nc * ns
    n = idx.shape[0]
    v = table.shape[0]
    blocks = v // 128
    bpw = blocks // nw
    n_extra = blocks - bpw * nw + 1
    cpw = (bpw * 128) // _CH
    nvmax = n // 16
    mesh = plsc.VectorSubcoreMesh(core_axis_name="c", subcore_axis_name="s")
    tableT = table.T

    @functools.partial(
        pl.kernel,
        mesh=mesh,
        out_type=jax.ShapeDtypeStruct((n_sessions * _LP, 2 * _D), jnp.float32),
        scratch_types=[
            pltpu.VMEM((n,), jnp.int32),
            pltpu.VMEM((n + 16,), jnp.int32),
            pltpu.VMEM((n + 16,), jnp.int32),
            pltpu.VMEM((2 * _D, _CH), jnp.float32),
            pltpu.VMEM((16, 2 * _D), jnp.float32),
            pltpu.VMEM((16,), jnp.int32),
            pltpu.SemaphoreType.DMA,
            pltpu.SemaphoreType.DMA,
            pltpu.SemaphoreType.DMA,
        ],
        compiler_params=pltpu.CompilerParams(needs_layout_passes=False),
    )
    def k(idx_hbm, tT_hbm, out_hbm, idx_v, wl_r, wl_op, cbuf, rstage, pstage,
          semA, semB, semS):
        wid = lax.axis_index("s") * nc + lax.axis_index("c")
        i16 = lax.iota(jnp.int32, 16)
        dummy = (jnp.int32(n_sessions - 16) + i16) * _LP + _L

        lo = wid * (bpw * 128)
        hi = lo + bpw * 128
        elo = jnp.where(wid < n_extra, (blocks - n_extra + 1 + wid) * 128,
                        jnp.int32(0))
        ehi = jnp.where(wid < n_extra, elo + 128, jnp.int32(0))

        def fetch(c, par, sem):
            lane0 = pl.multiple_of(lo + lax.rem(c, cpw) * _CH, 128)
            for g in range(8):
                pltpu.async_copy(
                    tT_hbm.at[pl.ds(g * 8, 8), pl.ds(lane0, _CH)],
                    cbuf.at[pl.ds(par * _D + g * 8, 8), :],
                    sem,
                )

        fetch(0, 0, semA)
        fetch(1, 1, semB)
        pltpu.sync_copy(idx_hbm, idx_v)
        pstage[pl.ds(0, 16)] = dummy

        def scan(i, cnt):
            rv = idx_v[pl.ds(16 * i, 16)]
            posv = 16 * i + i16
            sess = posv // _L
            opv = sess * _LP + (posv - sess * _L)
            m = ((rv >= lo) & (rv < hi)) | ((rv >= elo) & (rv < ehi))
            plsc.store_compressed(wl_r.at[pl.ds(cnt, 16)], rv, mask=m)
            plsc.store_compressed(wl_op.at[pl.ds(cnt, 16)], opv, mask=m)
            return cnt + plsc.all_reduce_population_count(m)[0]

        cnt = lax.fori_loop(0, nvmax, scan, jnp.int32(0))
        wl_r[pl.ds(cnt, 16)] = jnp.full((16,), _SENT, jnp.int32)
        nv = (cnt + 15) // 16

        def flush():
            pltpu.async_copy(rstage, out_hbm.at[pstage], semS).wait()
            pstage[pl.ds(0, 16)] = dummy

        def process_range(c_lo, span, pbase, bc):
            def per_vreg(vi, bc_):
                rv = wl_r[pl.ds(16 * vi, 16)]
                m0 = (rv >= c_lo) & (rv < c_lo + span)

                def more(st):
                    m_, b_ = st
                    return plsc.all_reduce_population_count(m_)[0] > 0

                def take(st):
                    m_, b_ = st
                    lane = plsc.all_reduce_ffs(m_)[0]
                    ii = 16 * vi + lane
                    r = plsc.load_gather(wl_r, [jnp.broadcast_to(ii, (16,))])[0]
                    op = plsc.load_gather(wl_op, [jnp.broadcast_to(ii, (16,))])[0]
                    col = jnp.broadcast_to(r - c_lo, (16,))
                    slot = lax.rem(b_, 16)
                    sl16 = jnp.broadcast_to(slot, (16,))
                    for g in range(_D // 16):
                        vals = plsc.load_gather(cbuf, [pbase + g * 16 + i16, col])
                        plsc.store_scatter(rstage, [sl16, g * 16 + i16], vals)
                    plsc.store_scatter(pstage, [sl16],
                                       jnp.broadcast_to(op, (16,)),
                                       mask=i16 == 0)
                    b_ = b_ + 1

                    @pl.when(lax.rem(b_, 16) == 0)
                    def _():
                        flush()

                    return m_ & (i16 != lane), b_

                _, bc_ = lax.while_loop(more, take, (m0, bc_))
                return bc_

            return lax.fori_loop(0, nv, per_vreg, bc)

        def drain(sem):
            pltpu.make_async_copy(
                tT_hbm.at[:, pl.ds(0, _CH)],
                cbuf.at[pl.ds(0, _D), :],
                sem,
            ).wait()

        def dbl(t, bc):
            c0 = 2 * t
            drain(semA)
            bc = process_range(lo + c0 * _CH, _CH, 0, bc)
            fetch(c0 + 2, 0, semA)
            drain(semB)
            bc = process_range(lo + (c0 + 1) * _CH, _CH, _D, bc)
            fetch(c0 + 3, 1, semB)
            return bc

        bc = lax.fori_loop(0, cpw // 2, dbl, jnp.int32(0))
        drain(semA)
        drain(semB)

        if cpw % 2 == 1:
            pltpu.sync_copy(
                tT_hbm.at[:, pl.ds(pl.multiple_of(lo + (cpw - 1) * _CH, 128),
                                   _CH)],
                cbuf.at[pl.ds(0, _D), :],
            )
            bc = process_range(lo + (cpw - 1) * _CH, _CH, 0, bc)

        def extra(e, bc_):
            lane0 = pl.multiple_of(elo, 128)
            pltpu.sync_copy(
                tT_hbm.at[:, pl.ds(lane0, 128)],
                cbuf.at[pl.ds(0, _D), pl.ds(0, 128)],
            )
            return process_range(elo, 128, 0, bc_)

        bc = lax.fori_loop(0, jnp.where(wid < n_extra, 1, 0), extra, bc)
        flush()

    return k(idx, tableT)


def _attn_block(blk_ref, adj_ref, aa_ref, o_ref):
    h = blk_ref[:, 0:_L, 0:_D]
    adj = adj_ref[...]
    aa = aa_ref[...]
    dn = (((2,), (2,)), ((0,), (0,)))
    es = []
    for kk in range(4):
        q = h * aa[kk][None, None, :]
        e = lax.dot_general(q, h, dn, preferred_element_type=jnp.float32)
        es.append(jnp.where(e > 0, e, _ALPHA * e))
    neg = jnp.float32(-9e15)
    alpha = jnp.where(adj == 1, es[0], neg)
    alpha = jnp.where(adj == 2, es[1], alpha)
    alpha = jnp.where(adj == 3, es[2], alpha)
    alpha = jnp.where(adj == 4, es[3], alpha)
    m = jnp.max(alpha, axis=-1, keepdims=True)
    ex = jnp.exp(alpha - m)
    p = ex / jnp.sum(ex, axis=-1, keepdims=True)
    dn2 = (((2,), (1,)), ((0,), (0,)))
    o_ref[...] = lax.dot_general(p, h, dn2, preferred_element_type=jnp.float32)


def kernel(inputs, adj, mask_item, item, embedding, a_0, a_1, a_2, a_3, bias):
    b, l = inputs.shape
    hp_flat = _gather_rows(embedding, inputs.reshape(-1), b)
    hp = hp_flat.reshape(b, _LP, 2 * _D)
    aa = jnp.concatenate([a_0, a_1, a_2, a_3], axis=1).T
    out = pl.pallas_call(
        _attn_block,
        grid=(b // _NB,),
        in_specs=[
            pl.BlockSpec((_NB, _LP, 2 * _D), lambda i: (i, 0, 0)),
            pl.BlockSpec((_NB, l, l), lambda i: (i, 0, 0)),
            pl.BlockSpec((4, _D), lambda i: (0, 0)),
        ],
        out_specs=pl.BlockSpec((_NB, l, _D), lambda i: (i, 0, 0)),
        out_shape=jax.ShapeDtypeStruct((b, l, _D), jnp.float32),
    )(hp, adj, aa)
    return (out, jnp.float32(0.0))


# speedup vs baseline: 1.0421x; 1.0421x over previous
import functools

import jax
import jax.numpy as jnp
from jax import lax
from jax.experimental import pallas as pl
from jax.experimental.pallas import tpu as pltpu
from jax.experimental.pallas import tpu_sc as plsc

_D = 64
_L = 20
_LP = 24
_ALPHA = 0.2
_NB = 64
_CH = 512
_SENT = 0x7FFFFFFF


def _gather_rows(table, idx, n_sessions):
    info = plsc.get_sparse_core_info()
    nc, ns = info.num_cores, info.num_subcores
    nw = nc * ns
    n = idx.shape[0]
    v = table.shape[0]
    blocks = v // 128
    bpw = blocks // nw
    n_extra = blocks - bpw * nw + 1
    cpw = (bpw * 128) // _CH
    nvmax = n // 16
    mesh = plsc.VectorSubcoreMesh(core_axis_name="c", subcore_axis_name="s")
    tableT = table.T

    @functools.partial(
        pl.kernel,
        mesh=mesh,
        out_type=jax.ShapeDtypeStruct((n_sessions * _LP, 2 * _D), jnp.float32),
        scratch_types=[
            pltpu.VMEM((n,), jnp.int32),
            pltpu.VMEM((n + 16,), jnp.int32),
            pltpu.VMEM((n + 16,), jnp.int32),
            pltpu.VMEM((2 * _D, _CH), jnp.float32),
            pltpu.VMEM((16, 2 * _D), jnp.float32),
            pltpu.VMEM((16,), jnp.int32),
            pltpu.SemaphoreType.DMA,
            pltpu.SemaphoreType.DMA,
            pltpu.SemaphoreType.DMA,
        ],
        compiler_params=pltpu.CompilerParams(needs_layout_passes=False),
    )
    def k(idx_hbm, tT_hbm, out_hbm, idx_v, wl_r, wl_op, cbuf, rstage, pstage,
          semA, semB, semS):
        wid = lax.axis_index("s") * nc + lax.axis_index("c")
        i16 = lax.iota(jnp.int32, 16)
        dummy = (jnp.int32(n_sessions - 16) + i16) * _LP + _L

        lo = wid * (bpw * 128)
        hi = lo + bpw * 128
        elo = jnp.where(wid < n_extra, (blocks - n_extra + 1 + wid) * 128,
                        jnp.int32(0))
        ehi = jnp.where(wid < n_extra, elo + 128, jnp.int32(0))

        def fetch(c, par, sem):
            lane0 = pl.multiple_of(lo + lax.rem(c, cpw) * _CH, 128)
            pltpu.async_copy(
                tT_hbm.at[:, pl.ds(lane0, _CH)],
                cbuf.at[pl.ds(par * _D, _D), :],
                sem,
            )

        fetch(0, 0, semA)
        fetch(1, 1, semB)
        pltpu.sync_copy(idx_hbm, idx_v)
        pstage[pl.ds(0, 16)] = dummy

        def scan(i, cnt):
            rv = idx_v[pl.ds(16 * i, 16)]
            posv = 16 * i + i16
            sess = posv // _L
            opv = sess * _LP + (posv - sess * _L)
            m = ((rv >= lo) & (rv < hi)) | ((rv >= elo) & (rv < ehi))
            plsc.store_compressed(wl_r.at[pl.ds(cnt, 16)], rv, mask=m)
            plsc.store_compressed(wl_op.at[pl.ds(cnt, 16)], opv, mask=m)
            return cnt + plsc.all_reduce_population_count(m)[0]

        cnt = lax.fori_loop(0, nvmax, scan, jnp.int32(0))
        wl_r[pl.ds(cnt, 16)] = jnp.full((16,), _SENT, jnp.int32)
        nv = (cnt + 15) // 16

        def flush():
            pltpu.async_copy(rstage, out_hbm.at[pstage], semS).wait()
            pstage[pl.ds(0, 16)] = dummy

        def process_range(c_lo, span, pbase, bc):
            def per_vreg(vi, bc_):
                rv = wl_r[pl.ds(16 * vi, 16)]
                m0 = (rv >= c_lo) & (rv < c_lo + span)

                def more(st):
                    m_, b_ = st
                    return plsc.all_reduce_population_count(m_)[0] > 0

                def take(st):
                    m_, b_ = st
                    lane = plsc.all_reduce_ffs(m_)[0]
                    ii = 16 * vi + lane
                    r = plsc.load_gather(wl_r, [jnp.broadcast_to(ii, (16,))])[0]
                    op = plsc.load_gather(wl_op, [jnp.broadcast_to(ii, (16,))])[0]
                    col = jnp.broadcast_to(r - c_lo, (16,))
                    slot = lax.rem(b_, 16)
                    sl16 = jnp.broadcast_to(slot, (16,))
                    for g in range(_D // 16):
                        vals = plsc.load_gather(cbuf, [pbase + g * 16 + i16, col])
                        plsc.store_scatter(rstage, [sl16, g * 16 + i16], vals)
                    plsc.store_scatter(pstage, [sl16],
                                       jnp.broadcast_to(op, (16,)),
                                       mask=i16 == 0)
                    b_ = b_ + 1

                    @pl.when(lax.rem(b_, 16) == 0)
                    def _():
                        flush()

                    return m_ & (i16 != lane), b_

                _, bc_ = lax.while_loop(more, take, (m0, bc_))
                return bc_

            return lax.fori_loop(0, nv, per_vreg, bc)

        def drain(sem):
            pltpu.make_async_copy(
                tT_hbm.at[:, pl.ds(0, _CH)],
                cbuf.at[pl.ds(0, _D), :],
                sem,
            ).wait()

        def dbl(t, bc):
            c0 = 2 * t
            drain(semA)
            bc = process_range(lo + c0 * _CH, _CH, 0, bc)
            fetch(c0 + 2, 0, semA)
            drain(semB)
            bc = process_range(lo + (c0 + 1) * _CH, _CH, _D, bc)
            fetch(c0 + 3, 1, semB)
            return bc

        bc = lax.fori_loop(0, cpw // 2, dbl, jnp.int32(0))
        drain(semA)
        drain(semB)

        if cpw % 2 == 1:
            pltpu.sync_copy(
                tT_hbm.at[:, pl.ds(pl.multiple_of(lo + (cpw - 1) * _CH, 128),
                                   _CH)],
                cbuf.at[pl.ds(0, _D), :],
            )
            bc = process_range(lo + (cpw - 1) * _CH, _CH, 0, bc)

        def extra(e, bc_):
            lane0 = pl.multiple_of(elo, 128)
            pltpu.sync_copy(
                tT_hbm.at[:, pl.ds(lane0, 128)],
                cbuf.at[pl.ds(0, _D), pl.ds(0, 128)],
            )
            return process_range(elo, 128, 0, bc_)

        bc = lax.fori_loop(0, jnp.where(wid < n_extra, 1, 0), extra, bc)
        flush()

    return k(idx, tableT)


def _attn_block(blk_ref, adj_ref, aa_ref, o_ref):
    h = blk_ref[:, 0:_L, 0:_D]
    adj = adj_ref[...]
    aa = aa_ref[...]
    dn = (((2,), (2,)), ((0,), (0,)))
    es = []
    for kk in range(4):
        q = h * aa[kk][None, None, :]
        e = lax.dot_general(q, h, dn, preferred_element_type=jnp.float32)
        es.append(jnp.where(e > 0, e, _ALPHA * e))
    neg = jnp.float32(-9e15)
    alpha = jnp.where(adj == 1, es[0], neg)
    alpha = jnp.where(adj == 2, es[1], alpha)
    alpha = jnp.where(adj == 3, es[2], alpha)
    alpha = jnp.where(adj == 4, es[3], alpha)
    m = jnp.max(alpha, axis=-1, keepdims=True)
    ex = jnp.exp(alpha - m)
    p = ex / jnp.sum(ex, axis=-1, keepdims=True)
    dn2 = (((2,), (1,)), ((0,), (0,)))
    o_ref[...] = lax.dot_general(p, h, dn2, preferred_element_type=jnp.float32)


def kernel(inputs, adj, mask_item, item, embedding, a_0, a_1, a_2, a_3, bias):
    b, l = inputs.shape
    hp_flat = _gather_rows(embedding, inputs.reshape(-1), b)
    hp = hp_flat.reshape(b, _LP, 2 * _D)
    aa = jnp.concatenate([a_0, a_1, a_2, a_3], axis=1).T
    out = pl.pallas_call(
        _attn_block,
        grid=(b // _NB,),
        in_specs=[
            pl.BlockSpec((_NB, _LP, 2 * _D), lambda i: (i, 0, 0)),
            pl.BlockSpec((_NB, l, l), lambda i: (i, 0, 0)),
            pl.BlockSpec((4, _D), lambda i: (0, 0)),
        ],
        out_specs=pl.BlockSpec((_NB, l, _D), lambda i: (i, 0, 0)),
        out_shape=jax.ShapeDtypeStruct((b, l, _D), jnp.float32),
    )(hp, adj, aa)
    return (out, jnp.float32(0.0))


# speedup vs baseline: 1.0558x; 1.0131x over previous
import functools

import jax
import jax.numpy as jnp
from jax import lax
from jax.experimental import pallas as pl
from jax.experimental.pallas import tpu as pltpu
from jax.experimental.pallas import tpu_sc as plsc

_D = 64
_L = 20
_LP = 24
_ALPHA = 0.2
_NB = 128
_CH = 512
_SENT = 0x7FFFFFFF


def _gather_rows(table, idx, n_sessions):
    info = plsc.get_sparse_core_info()
    nc, ns = info.num_cores, info.num_subcores
    nw = nc * ns
    n = idx.shape[0]
    v = table.shape[0]
    blocks = v // 128
    bpw = blocks // nw
    n_extra = blocks - bpw * nw + 1
    cpw = (bpw * 128) // _CH
    nvmax = n // 16
    mesh = plsc.VectorSubcoreMesh(core_axis_name="c", subcore_axis_name="s")
    tableT = table.T

    @functools.partial(
        pl.kernel,
        mesh=mesh,
        out_type=jax.ShapeDtypeStruct((n_sessions * _LP, 2 * _D), jnp.float32),
        scratch_types=[
            pltpu.VMEM((n,), jnp.int32),
            pltpu.VMEM((n + 16,), jnp.int32),
            pltpu.VMEM((n + 16,), jnp.int32),
            pltpu.VMEM((2 * _D, _CH), jnp.float32),
            pltpu.VMEM((16, 2 * _D), jnp.float32),
            pltpu.VMEM((16,), jnp.int32),
            pltpu.SemaphoreType.DMA,
            pltpu.SemaphoreType.DMA,
            pltpu.SemaphoreType.DMA,
        ],
        compiler_params=pltpu.CompilerParams(needs_layout_passes=False),
    )
    def k(idx_hbm, tT_hbm, out_hbm, idx_v, wl_r, wl_op, cbuf, rstage, pstage,
          semA, semB, semS):
        wid = lax.axis_index("s") * nc + lax.axis_index("c")
        i16 = lax.iota(jnp.int32, 16)
        dummy = (jnp.int32(n_sessions - 16) + i16) * _LP + _L

        lo = wid * (bpw * 128)
        hi = lo + bpw * 128
        elo = jnp.where(wid < n_extra, (blocks - n_extra + 1 + wid) * 128,
                        jnp.int32(0))
        ehi = jnp.where(wid < n_extra, elo + 128, jnp.int32(0))

        def fetch(c, par, sem):
            lane0 = pl.multiple_of(lo + lax.rem(c, cpw) * _CH, 128)
            pltpu.async_copy(
                tT_hbm.at[:, pl.ds(lane0, _CH)],
                cbuf.at[pl.ds(par * _D, _D), :],
                sem,
            )

        fetch(0, 0, semA)
        fetch(1, 1, semB)
        pltpu.sync_copy(idx_hbm, idx_v)
        pstage[pl.ds(0, 16)] = dummy

        def scan(i, cnt):
            rv = idx_v[pl.ds(16 * i, 16)]
            posv = 16 * i + i16
            sess = posv // _L
            opv = sess * _LP + (posv - sess * _L)
            m = ((rv >= lo) & (rv < hi)) | ((rv >= elo) & (rv < ehi))
            plsc.store_compressed(wl_r.at[pl.ds(cnt, 16)], rv, mask=m)
            plsc.store_compressed(wl_op.at[pl.ds(cnt, 16)], opv, mask=m)
            return cnt + plsc.all_reduce_population_count(m)[0]

        cnt = lax.fori_loop(0, nvmax, scan, jnp.int32(0))
        wl_r[pl.ds(cnt, 16)] = jnp.full((16,), _SENT, jnp.int32)
        nv = (cnt + 15) // 16

        def flush():
            pltpu.async_copy(rstage, out_hbm.at[pstage], semS).wait()
            pstage[pl.ds(0, 16)] = dummy

        def process_range(c_lo, span, pbase, bc):
            def per_vreg(vi, bc_):
                rv = wl_r[pl.ds(16 * vi, 16)]
                m0 = (rv >= c_lo) & (rv < c_lo + span)

                def more(st):
                    m_, b_ = st
                    return plsc.all_reduce_population_count(m_)[0] > 0

                def take(st):
                    m_, b_ = st
                    lane = plsc.all_reduce_ffs(m_)[0]
                    ii = 16 * vi + lane
                    r = plsc.load_gather(wl_r, [jnp.broadcast_to(ii, (16,))])[0]
                    op = plsc.load_gather(wl_op, [jnp.broadcast_to(ii, (16,))])[0]
                    col = jnp.broadcast_to(r - c_lo, (16,))
                    slot = lax.rem(b_, 16)
                    sl16 = jnp.broadcast_to(slot, (16,))
                    for g in range(_D // 16):
                        vals = plsc.load_gather(cbuf, [pbase + g * 16 + i16, col])
                        plsc.store_scatter(rstage, [sl16, g * 16 + i16], vals)
                    plsc.store_scatter(pstage, [sl16],
                                       jnp.broadcast_to(op, (16,)),
                                       mask=i16 == 0)
                    b_ = b_ + 1

                    @pl.when(lax.rem(b_, 16) == 0)
                    def _():
                        flush()

                    return m_ & (i16 != lane), b_

                _, bc_ = lax.while_loop(more, take, (m0, bc_))
                return bc_

            return lax.fori_loop(0, nv, per_vreg, bc)

        def drain(sem):
            pltpu.make_async_copy(
                tT_hbm.at[:, pl.ds(0, _CH)],
                cbuf.at[pl.ds(0, _D), :],
                sem,
            ).wait()

        def dbl(t, bc):
            c0 = 2 * t
            drain(semA)
            bc = process_range(lo + c0 * _CH, _CH, 0, bc)
            fetch(c0 + 2, 0, semA)
            drain(semB)
            bc = process_range(lo + (c0 + 1) * _CH, _CH, _D, bc)
            fetch(c0 + 3, 1, semB)
            return bc

        bc = lax.fori_loop(0, cpw // 2, dbl, jnp.int32(0))
        drain(semA)
        drain(semB)

        if cpw % 2 == 1:
            pltpu.sync_copy(
                tT_hbm.at[:, pl.ds(pl.multiple_of(lo + (cpw - 1) * _CH, 128),
                                   _CH)],
                cbuf.at[pl.ds(0, _D), :],
            )
            bc = process_range(lo + (cpw - 1) * _CH, _CH, 0, bc)

        def extra(e, bc_):
            lane0 = pl.multiple_of(elo, 128)
            pltpu.sync_copy(
                tT_hbm.at[:, pl.ds(lane0, 128)],
                cbuf.at[pl.ds(0, _D), pl.ds(0, 128)],
            )
            return process_range(elo, 128, 0, bc_)

        bc = lax.fori_loop(0, jnp.where(wid < n_extra, 1, 0), extra, bc)
        flush()

    return k(idx, tableT)


def _attn_block(blk_ref, adj_ref, aa_ref, o_ref):
    h = blk_ref[:, 0:_L, 0:_D]
    adj = adj_ref[...]
    aa = aa_ref[...]
    dn = (((2,), (2,)), ((0,), (0,)))
    es = []
    for kk in range(4):
        q = h * aa[kk][None, None, :]
        e = lax.dot_general(q, h, dn, preferred_element_type=jnp.float32)
        es.append(jnp.where(e > 0, e, _ALPHA * e))
    neg = jnp.float32(-9e15)
    alpha = jnp.where(adj == 1, es[0], neg)
    alpha = jnp.where(adj == 2, es[1], alpha)
    alpha = jnp.where(adj == 3, es[2], alpha)
    alpha = jnp.where(adj == 4, es[3], alpha)
    m = jnp.max(alpha, axis=-1, keepdims=True)
    ex = jnp.exp(alpha - m)
    p = ex / jnp.sum(ex, axis=-1, keepdims=True)
    dn2 = (((2,), (1,)), ((0,), (0,)))
    o_ref[...] = lax.dot_general(p, h, dn2, preferred_element_type=jnp.float32)


def kernel(inputs, adj, mask_item, item, embedding, a_0, a_1, a_2, a_3, bias):
    b, l = inputs.shape
    hp_flat = _gather_rows(embedding, inputs.reshape(-1), b)
    hp = hp_flat.reshape(b, _LP, 2 * _D)
    aa = jnp.concatenate([a_0, a_1, a_2, a_3], axis=1).T
    out = pl.pallas_call(
        _attn_block,
        grid=(b // _NB,),
        in_specs=[
            pl.BlockSpec((_NB, _LP, 2 * _D), lambda i: (i, 0, 0)),
            pl.BlockSpec((_NB, l, l), lambda i: (i, 0, 0)),
            pl.BlockSpec((4, _D), lambda i: (0, 0)),
        ],
        out_specs=pl.BlockSpec((_NB, l, _D), lambda i: (i, 0, 0)),
        out_shape=jax.ShapeDtypeStruct((b, l, _D), jnp.float32),
    )(hp, adj, aa)
    return (out, jnp.float32(0.0))
